# baseline (speedup 1.0000x reference)
import functools

import jax
import jax.numpy as jnp
from jax import lax
from jax.experimental import pallas as pl
from jax.experimental.pallas import tpu as pltpu
from jax.experimental.pallas import tpu_sc as plsc


def _build_sc_add(B, S, D, M):
    info = plsc.get_sparse_core_info()
    NC, NS, L = info.num_cores, info.num_subcores, info.num_lanes
    NW = NC * NS
    assert S % NW == 0
    rows_per_w = S // NW
    R = 32
    n_chunks = rows_per_w // R
    assert rows_per_w % R == 0 and D % L == 0

    mesh = plsc.VectorSubcoreMesh(core_axis_name="c", subcore_axis_name="s")

    @functools.partial(
        pl.kernel,
        out_type=jax.ShapeDtypeStruct((B * S, D), jnp.float32),
        mesh=mesh,
        scratch_types=[
            pltpu.VMEM((rows_per_w,), jnp.int32),
            pltpu.VMEM((R, D), jnp.float32),
            pltpu.VMEM((R, D), jnp.float32),
            pltpu.SemaphoreType.DMA,
        ],
    )
    def sc_add(x_hbm, pos_hbm, idx_hbm, out_hbm, idxbuf, posbuf, xbuf, sem):
        c = lax.axis_index("c")
        s = lax.axis_index("s")
        wid = s * NC + c
        base = wid * rows_per_w
        pltpu.sync_copy(idx_hbm.at[pl.ds(base, rows_per_w)], idxbuf)

        def chunk_body(k, carry):
            start = base + k * R
            pltpu.async_copy(
                pos_hbm.at[idxbuf.at[pl.ds(k * R, R)]], posbuf, sem
            ).wait()
            for b in range(B):
                row0 = b * S + start
                pltpu.sync_copy(x_hbm.at[pl.ds(row0, R), :], xbuf)

                def row_body(i, carry2):
                    for j in range(D // L):
                        sl = pl.ds(j * L, L)
                        plsc.addupdate(xbuf.at[i, sl], posbuf[i, sl])
                    return carry2

                lax.fori_loop(0, R, row_body, 0)
                pltpu.sync_copy(xbuf, out_hbm.at[pl.ds(row0, R), :])
            return carry

        lax.fori_loop(0, n_chunks, chunk_body, 0)

    return sc_add


@jax.jit
def kernel(x, pos_table, offset):
    B, S, D = x.shape
    M = pos_table.shape[0]
    positions = jnp.clip(
        jnp.asarray(offset, jnp.int32) + jnp.arange(S, dtype=jnp.int32), 0, M - 1
    )
    x2 = x.reshape(B * S, D)
    out = _build_sc_add(B, S, D, M)(x2, pos_table, positions)
    return out.reshape(B, S, D)
